# Initial kernel scaffold; baseline (speedup 1.0000x reference)
import jax
import jax.numpy as jnp
from jax.experimental import pallas as pl


def kernel(edge_index, node_types, node_emb, type_emb, W_l0, b_l0, W_r0, bn_g0, bn_b0, W_l1, b_l1, W_r1, bn_g1, bn_b1, W_l2, b_l2, W_r2):
    raise NotImplementedError("write your pallas kernel here")


# speedup vs baseline: 3.1352x; 3.1352x over previous
import functools
import numpy as np
import jax
import jax.numpy as jnp
from jax import lax
from jax.experimental import pallas as pl
from jax.experimental.pallas import tpu as pltpu
from jax.experimental.pallas import tpu_sc as plsc

N = 10000
E = 160000
NP = 10240
EB = 128
RPT = 80
EPT = RPT * EB
E_PAD = 16 * EPT
BM = 2000
NCORES = 2
NSUB = 16

_mesh = lambda: plsc.VectorSubcoreMesh(
    core_axis_name="c", subcore_axis_name="s", num_cores=NCORES, num_subcores=NSUB)


@functools.cache
def _make_segsum(nchunks):
  def body(src_hbm, dst_hbm, *rest):
    feats = rest[:nchunks]
    outs = rest[nchunks:2 * nchunks]
    srcv, dstv, gbuf, accum = rest[2 * nchunks:]
    core = lax.axis_index("c")
    t = lax.axis_index("s")
    pltpu.sync_copy(src_hbm.at[pl.ds(t * EPT, EPT)], srcv)
    pltpu.sync_copy(dst_hbm.at[pl.ds(t * RPT, RPT)], dstv)

    for step in range(nchunks // 2):
      def zrow(i, _):
        for g in range(8):
          gbuf[i, pl.ds(g * 16, 16)] = jnp.zeros((16,), jnp.float32)
        return 0
      lax.fori_loop(0, EB, zrow, 0)
      for k5 in range(5):
        pltpu.sync_copy(gbuf, accum.at[pl.ds(t * 640 + k5 * EB, EB)])
      plsc.subcore_barrier()

      def bbody(b, _):
        for k in range(2):
          @pl.when(core == k)
          def _():
            pltpu.sync_copy(feats[2 * step + k].at[srcv.at[pl.ds(b * EB, EB)]],
                            gbuf)
            pltpu.sync_copy(gbuf, accum.at[dstv.at[b]], add=True)
        return 0
      lax.fori_loop(0, RPT, bbody, 0)
      plsc.subcore_barrier()

      for k in range(2):
        @pl.when(core == k)
        def _():
          pltpu.sync_copy(accum.at[pl.ds(t * 640, 640)],
                          outs[2 * step + k].at[pl.ds(t * 640, 640)])
      if step + 1 < nchunks // 2:
        plsc.subcore_barrier()

  return pl.kernel(
      body,
      out_type=[jax.ShapeDtypeStruct((NP, 128), jnp.float32)] * nchunks,
      mesh=_mesh(),
      scratch_types=[
          pltpu.VMEM((EPT,), jnp.int32),
          pltpu.VMEM((RPT, EB), jnp.int32),
          pltpu.VMEM((EB, 128), jnp.float32),
          pltpu.VMEM_SHARED((NP, 128), jnp.float32),
      ])


@functools.cache
def _make_deg():
  def body(dst_hbm, deg_hbm, dstv, degv, degout, vbuf, sp):
    core = lax.axis_index("c")
    t = lax.axis_index("s")
    pltpu.sync_copy(dst_hbm.at[pl.ds(t * RPT, RPT)], dstv)

    def z(i, _):
      degv[pl.ds(i * 16, 16)] = jnp.zeros((16,), jnp.float32)
      return 0
    lax.fori_loop(0, NP // 16, z, 0)

    ones = jnp.ones((16,), jnp.float32)
    def acc(r, _):
      for g in range(EB // 16):
        idx = dstv[r, pl.ds(g * 16, 16)]
        plsc.addupdate_scatter(degv, [idx], ones)
      return 0
    lax.fori_loop(0, RPT, acc, 0)

    pltpu.sync_copy(degv, sp.at[t])
    plsc.subcore_barrier()
    for r in range(16):
      pltpu.sync_copy(sp.at[r, pl.ds(t * 640, 640)], vbuf.at[r])

    def red(g, _):
      s = jnp.zeros((16,), jnp.float32)
      for r in range(16):
        s = s + vbuf[r, pl.ds(g * 16, 16)]
      degout[pl.ds(g * 16, 16)] = s
      return 0
    lax.fori_loop(0, 40, red, 0)

    @pl.when(core == 0)
    def _():
      pltpu.sync_copy(degout, deg_hbm.at[pl.ds(t * 640, 640)])

  return pl.kernel(
      body,
      out_type=jax.ShapeDtypeStruct((NP,), jnp.float32),
      mesh=_mesh(),
      compiler_params=pltpu.CompilerParams(needs_layout_passes=False),
      scratch_types=[
          pltpu.VMEM((RPT, EB), jnp.int32),
          pltpu.VMEM((NP,), jnp.float32),
          pltpu.VMEM((640,), jnp.float32),
          pltpu.VMEM((16, 640), jnp.float32),
          pltpu.VMEM_SHARED((16, NP), jnp.float32),
      ])


def _embed_body(types_ref, emb_ref, temb_ref, o0, o1):
  t = types_ref[...]
  oh = (t == lax.broadcasted_iota(jnp.int32, (1, 8), 1)).astype(jnp.float32)
  x = emb_ref[...] + jnp.dot(oh, temb_ref[...], preferred_element_type=jnp.float32)
  o0[...] = x[:, :128]
  o1[...] = x[:, 128:]


_embed = pl.pallas_call(
    _embed_body,
    grid=(N // BM,),
    in_specs=[
        pl.BlockSpec((BM, 1), lambda i: (i, 0)),
        pl.BlockSpec((BM, 256), lambda i: (i, 0)),
        pl.BlockSpec((8, 256), lambda i: (0, 0)),
    ],
    out_specs=[pl.BlockSpec((BM, 128), lambda i: (i, 0))] * 2,
    out_shape=[jax.ShapeDtypeStruct((N, 128), jnp.float32)] * 2)


def _make_sage(cin, cout):
  dout = cout * 128
  def body(*refs):
    aggs = refs[0:cin]
    xs = refs[cin:2 * cin]
    dref, Wl, Wr, bl, g, bb = refs[2 * cin:2 * cin + 6]
    outs = refs[2 * cin + 6:]
    inv = 1.0 / jnp.maximum(dref[...], 1.0)
    acc = jnp.zeros((BM, dout), jnp.float32)
    for c in range(cin):
      acc = acc + jnp.dot(aggs[c][...] * inv, Wl[c * 128:(c + 1) * 128, :],
                          preferred_element_type=jnp.float32)
      acc = acc + jnp.dot(xs[c][...], Wr[c * 128:(c + 1) * 128, :],
                          preferred_element_type=jnp.float32)
    s = g[...] * np.float32(1.0 / np.sqrt(1.0 + 1e-5))
    h = jnp.maximum(acc * s + (bl[...] * s + bb[...]), 0.0)
    for c in range(cout):
      outs[c][...] = h[:, c * 128:(c + 1) * 128]

  din = cin * 128
  return pl.pallas_call(
      body,
      grid=(N // BM,),
      in_specs=(
          [pl.BlockSpec((BM, 128), lambda i: (i, 0))] * (2 * cin) +
          [pl.BlockSpec((BM, 1), lambda i: (i, 0)),
           pl.BlockSpec((din, dout), lambda i: (0, 0)),
           pl.BlockSpec((din, dout), lambda i: (0, 0)),
           pl.BlockSpec((1, dout), lambda i: (0, 0)),
           pl.BlockSpec((1, dout), lambda i: (0, 0)),
           pl.BlockSpec((1, dout), lambda i: (0, 0))]),
      out_specs=[pl.BlockSpec((BM, 128), lambda i: (i, 0))] * cout,
      out_shape=[jax.ShapeDtypeStruct((N, 128), jnp.float32)] * cout)


def _l2_body(x0, x1, x2, x3, Wl, Wr, bl, u0, u1, v):
  xs = (x0, x1, x2, x3)
  u = jnp.zeros((BM, 256), jnp.float32)
  vv = jnp.zeros((BM, 256), jnp.float32)
  for c in range(4):
    u = u + jnp.dot(xs[c][...], Wl[c * 128:(c + 1) * 128, :],
                    preferred_element_type=jnp.float32)
    vv = vv + jnp.dot(xs[c][...], Wr[c * 128:(c + 1) * 128, :],
                      preferred_element_type=jnp.float32)
  vv = vv + bl[...]
  u0[...] = u[:, :128]
  u1[...] = u[:, 128:]
  v[...] = vv


_l2 = pl.pallas_call(
    _l2_body,
    grid=(N // BM,),
    in_specs=(
        [pl.BlockSpec((BM, 128), lambda i: (i, 0))] * 4 +
        [pl.BlockSpec((512, 256), lambda i: (0, 0))] * 2 +
        [pl.BlockSpec((1, 256), lambda i: (0, 0))]),
    out_specs=[pl.BlockSpec((BM, 128), lambda i: (i, 0))] * 2 +
              [pl.BlockSpec((BM, 256), lambda i: (i, 0))],
    out_shape=[jax.ShapeDtypeStruct((N, 128), jnp.float32)] * 2 +
              [jax.ShapeDtypeStruct((N, 256), jnp.float32)])


def _fin_body(a0, a1, dref, vref, o):
  inv = 1.0 / jnp.maximum(dref[...], 1.0)
  o[...] = jnp.concatenate([a0[...] * inv, a1[...] * inv], axis=1) + vref[...]


_fin = pl.pallas_call(
    _fin_body,
    grid=(N // BM,),
    in_specs=[
        pl.BlockSpec((BM, 128), lambda i: (i, 0)),
        pl.BlockSpec((BM, 128), lambda i: (i, 0)),
        pl.BlockSpec((BM, 1), lambda i: (i, 0)),
        pl.BlockSpec((BM, 256), lambda i: (i, 0)),
    ],
    out_specs=pl.BlockSpec((BM, 256), lambda i: (i, 0)),
    out_shape=jax.ShapeDtypeStruct((N, 256), jnp.float32))


_sage0 = _make_sage(2, 4)
_sage1 = _make_sage(4, 4)


def kernel(edge_index, node_types, node_emb, type_emb, W_l0, b_l0, W_r0,
           bn_g0, bn_b0, W_l1, b_l1, W_r1, bn_g1, bn_b1, W_l2, b_l2, W_r2):
  src = edge_index[0]
  dst = edge_index[1]
  pad = E_PAD - E
  src1d = jnp.concatenate([src, jnp.zeros((pad,), src.dtype)]).astype(jnp.int32)
  dst2d = jnp.concatenate([dst, jnp.full((pad,), N, dst.dtype)]).reshape(
      E_PAD // EB, EB).astype(jnp.int32)
  types2d = node_types.reshape(N, 1).astype(jnp.int32)

  x0, x1 = _embed(types2d, node_emb, type_emb)
  degf = _make_deg()(dst2d)
  dcol = degf[:N].reshape(N, 1)

  a0, a1 = _make_segsum(2)(src1d, dst2d, x0, x1)
  h = _sage0(a0, a1, x0, x1, dcol, W_l0, W_r0, b_l0.reshape(1, -1),
             bn_g0.reshape(1, -1), bn_b0.reshape(1, -1))
  b4 = _make_segsum(4)(src1d, dst2d, *h)
  h2 = _sage1(*b4, *h, dcol, W_l1, W_r1, b_l1.reshape(1, -1),
              bn_g1.reshape(1, -1), bn_b1.reshape(1, -1))
  u0, u1, v = _l2(*h2, W_l2, W_r2, b_l2.reshape(1, -1))
  c0, c1 = _make_segsum(2)(src1d, dst2d, u0, u1)
  return _fin(c0, c1, dcol, v)


# speedup vs baseline: 3.5410x; 1.1294x over previous
import functools
import numpy as np
import jax
import jax.numpy as jnp
from jax import lax
from jax.experimental import pallas as pl
from jax.experimental.pallas import tpu as pltpu
from jax.experimental.pallas import tpu_sc as plsc

N = 10000
E = 160000
NP = 10240
EB = 128
RPT = 80
EPT = RPT * EB
E_PAD = 16 * EPT
BM = 2000
NCORES = 2
NSUB = 16

_mesh = lambda: plsc.VectorSubcoreMesh(
    core_axis_name="c", subcore_axis_name="s", num_cores=NCORES, num_subcores=NSUB)


@functools.cache
def _make_segsum(nchunks):
  NG = RPT // 8

  def body(ix_hbm, *rest):
    feats = rest[:nchunks]
    outs = rest[nchunks:2 * nchunks]
    ibuf, gb0, gb1, accum, ssem0, ssem1, isem = rest[2 * nchunks:]
    gbufs = (gb0, gb1)
    ssems = (ssem0, ssem1)
    core = lax.axis_index("c")
    t = lax.axis_index("s")

    for step in range(nchunks // 2):
      def zrow(i, _):
        for gq in range(8):
          gb0[i, pl.ds(gq * 16, 16)] = jnp.zeros((16,), jnp.float32)
        return 0
      lax.fori_loop(0, EB, zrow, 0)
      for k5 in range(5):
        pltpu.sync_copy(gb0, accum.at[pl.ds(t * 640 + k5 * EB, EB)])
      plsc.subcore_barrier()

      for k in range(2):
        @pl.when(core == k)
        def _(step=step, k=k):
          feat = feats[2 * step + k]
          pltpu.sync_copy(ix_hbm.at[pl.ds(t * NG * 16, 16)], ibuf.at[0])

          def gloop(g, _):
            @pl.when(g > 0)
            def _wi():
              pltpu.make_async_copy(
                  ix_hbm.at[pl.ds(0, 16)], ibuf.at[0], isem).wait()
            p = lax.rem(g, 2)
            for jj in range(4):
              for kk in range(2):
                b = g * 8 + jj * 2 + kk
                r = jj * 2 + kk
                @pl.when(b >= 2)
                def _ws(kk=kk):
                  pltpu.make_async_copy(
                      gbufs[kk], accum.at[ibuf.at[0, 8]], ssems[kk]).wait()
                pltpu.sync_copy(feat.at[ibuf.at[p, r]], gbufs[kk])
                pltpu.async_copy(gbufs[kk], accum.at[ibuf.at[p, 8 + r]],
                                 ssems[kk], add=True)
              if jj == 0:
                @pl.when(g + 1 < NG)
                def _pf(g=g):
                  p2 = lax.rem(g + 1, 2)
                  pltpu.async_copy(
                      ix_hbm.at[pl.ds((t * NG + g + 1) * 16, 16)],
                      ibuf.at[p2], isem)
            return 0
          lax.fori_loop(0, NG, gloop, 0)
          pltpu.make_async_copy(
              gbufs[0], accum.at[ibuf.at[0, 8]], ssems[0]).wait()
          pltpu.make_async_copy(
              gbufs[1], accum.at[ibuf.at[0, 8]], ssems[1]).wait()
      plsc.subcore_barrier()

      for k in range(2):
        @pl.when(core == k)
        def _(step=step, k=k):
          pltpu.sync_copy(accum.at[pl.ds(t * 640, 640)],
                          outs[2 * step + k].at[pl.ds(t * 640, 640)])
      if step + 1 < nchunks // 2:
        plsc.subcore_barrier()

  return pl.kernel(
      body,
      out_type=[jax.ShapeDtypeStruct((NP, 128), jnp.float32)] * nchunks,
      mesh=_mesh(),
      scratch_types=[
          pltpu.VMEM((2, 16, EB), jnp.int32),
          pltpu.VMEM((EB, 128), jnp.float32),
          pltpu.VMEM((EB, 128), jnp.float32),
          pltpu.VMEM_SHARED((NP, 128), jnp.float32),
          pltpu.SemaphoreType.DMA,
          pltpu.SemaphoreType.DMA,
          pltpu.SemaphoreType.DMA,
      ])


@functools.cache
def _make_deg():
  def body(dst_hbm, deg_hbm, dstv, degv, degout, vbuf, sp):
    core = lax.axis_index("c")
    t = lax.axis_index("s")
    pltpu.sync_copy(dst_hbm.at[pl.ds(t * RPT, RPT)], dstv)

    def z(i, _):
      degv[pl.ds(i * 16, 16)] = jnp.zeros((16,), jnp.float32)
      return 0
    lax.fori_loop(0, NP // 16, z, 0)

    ones = jnp.ones((16,), jnp.float32)
    def acc(r, _):
      for g in range(EB // 16):
        idx = dstv[r, pl.ds(g * 16, 16)]
        plsc.addupdate_scatter(degv, [idx], ones)
      return 0
    lax.fori_loop(0, RPT, acc, 0)

    pltpu.sync_copy(degv, sp.at[t])
    plsc.subcore_barrier()
    for r in range(16):
      pltpu.sync_copy(sp.at[r, pl.ds(t * 640, 640)], vbuf.at[r])

    def red(g, _):
      s = jnp.zeros((16,), jnp.float32)
      for r in range(16):
        s = s + vbuf[r, pl.ds(g * 16, 16)]
      degout[pl.ds(g * 16, 16)] = s
      return 0
    lax.fori_loop(0, 40, red, 0)

    @pl.when(core == 0)
    def _():
      pltpu.sync_copy(degout, deg_hbm.at[pl.ds(t * 640, 640)])

  return pl.kernel(
      body,
      out_type=jax.ShapeDtypeStruct((NP,), jnp.float32),
      mesh=_mesh(),
      compiler_params=pltpu.CompilerParams(needs_layout_passes=False),
      scratch_types=[
          pltpu.VMEM((RPT, EB), jnp.int32),
          pltpu.VMEM((NP,), jnp.float32),
          pltpu.VMEM((640,), jnp.float32),
          pltpu.VMEM((16, 640), jnp.float32),
          pltpu.VMEM_SHARED((16, NP), jnp.float32),
      ])


def _embed_body(types_ref, emb_ref, temb_ref, o0, o1):
  t = types_ref[...]
  oh = (t == lax.broadcasted_iota(jnp.int32, (1, 8), 1)).astype(jnp.float32)
  x = emb_ref[...] + jnp.dot(oh, temb_ref[...], preferred_element_type=jnp.float32)
  o0[...] = x[:, :128]
  o1[...] = x[:, 128:]


_embed = pl.pallas_call(
    _embed_body,
    grid=(N // BM,),
    in_specs=[
        pl.BlockSpec((BM, 1), lambda i: (i, 0)),
        pl.BlockSpec((BM, 256), lambda i: (i, 0)),
        pl.BlockSpec((8, 256), lambda i: (0, 0)),
    ],
    out_specs=[pl.BlockSpec((BM, 128), lambda i: (i, 0))] * 2,
    out_shape=[jax.ShapeDtypeStruct((N, 128), jnp.float32)] * 2)


def _make_sage(cin, cout):
  dout = cout * 128
  def body(*refs):
    aggs = refs[0:cin]
    xs = refs[cin:2 * cin]
    dref, Wl, Wr, bl, g, bb = refs[2 * cin:2 * cin + 6]
    outs = refs[2 * cin + 6:]
    inv = 1.0 / jnp.maximum(dref[...], 1.0)
    acc = jnp.zeros((BM, dout), jnp.float32)
    for c in range(cin):
      acc = acc + jnp.dot(aggs[c][...] * inv, Wl[c * 128:(c + 1) * 128, :],
                          preferred_element_type=jnp.float32)
      acc = acc + jnp.dot(xs[c][...], Wr[c * 128:(c + 1) * 128, :],
                          preferred_element_type=jnp.float32)
    s = g[...] * np.float32(1.0 / np.sqrt(1.0 + 1e-5))
    h = jnp.maximum(acc * s + (bl[...] * s + bb[...]), 0.0)
    for c in range(cout):
      outs[c][...] = h[:, c * 128:(c + 1) * 128]

  din = cin * 128
  return pl.pallas_call(
      body,
      grid=(N // BM,),
      in_specs=(
          [pl.BlockSpec((BM, 128), lambda i: (i, 0))] * (2 * cin) +
          [pl.BlockSpec((BM, 1), lambda i: (i, 0)),
           pl.BlockSpec((din, dout), lambda i: (0, 0)),
           pl.BlockSpec((din, dout), lambda i: (0, 0)),
           pl.BlockSpec((1, dout), lambda i: (0, 0)),
           pl.BlockSpec((1, dout), lambda i: (0, 0)),
           pl.BlockSpec((1, dout), lambda i: (0, 0))]),
      out_specs=[pl.BlockSpec((BM, 128), lambda i: (i, 0))] * cout,
      out_shape=[jax.ShapeDtypeStruct((N, 128), jnp.float32)] * cout)


def _l2_body(x0, x1, x2, x3, Wl, Wr, bl, u0, u1, v):
  xs = (x0, x1, x2, x3)
  u = jnp.zeros((BM, 256), jnp.float32)
  vv = jnp.zeros((BM, 256), jnp.float32)
  for c in range(4):
    u = u + jnp.dot(xs[c][...], Wl[c * 128:(c + 1) * 128, :],
                    preferred_element_type=jnp.float32)
    vv = vv + jnp.dot(xs[c][...], Wr[c * 128:(c + 1) * 128, :],
                      preferred_element_type=jnp.float32)
  vv = vv + bl[...]
  u0[...] = u[:, :128]
  u1[...] = u[:, 128:]
  v[...] = vv


_l2 = pl.pallas_call(
    _l2_body,
    grid=(N // BM,),
    in_specs=(
        [pl.BlockSpec((BM, 128), lambda i: (i, 0))] * 4 +
        [pl.BlockSpec((512, 256), lambda i: (0, 0))] * 2 +
        [pl.BlockSpec((1, 256), lambda i: (0, 0))]),
    out_specs=[pl.BlockSpec((BM, 128), lambda i: (i, 0))] * 2 +
              [pl.BlockSpec((BM, 256), lambda i: (i, 0))],
    out_shape=[jax.ShapeDtypeStruct((N, 128), jnp.float32)] * 2 +
              [jax.ShapeDtypeStruct((N, 256), jnp.float32)])


def _fin_body(a0, a1, dref, vref, o):
  inv = 1.0 / jnp.maximum(dref[...], 1.0)
  o[...] = jnp.concatenate([a0[...] * inv, a1[...] * inv], axis=1) + vref[...]


_fin = pl.pallas_call(
    _fin_body,
    grid=(N // BM,),
    in_specs=[
        pl.BlockSpec((BM, 128), lambda i: (i, 0)),
        pl.BlockSpec((BM, 128), lambda i: (i, 0)),
        pl.BlockSpec((BM, 1), lambda i: (i, 0)),
        pl.BlockSpec((BM, 256), lambda i: (i, 0)),
    ],
    out_specs=pl.BlockSpec((BM, 256), lambda i: (i, 0)),
    out_shape=jax.ShapeDtypeStruct((N, 256), jnp.float32))


_sage0 = _make_sage(2, 4)
_sage1 = _make_sage(4, 4)


def kernel(edge_index, node_types, node_emb, type_emb, W_l0, b_l0, W_r0,
           bn_g0, bn_b0, W_l1, b_l1, W_r1, bn_g1, bn_b1, W_l2, b_l2, W_r2):
  src = edge_index[0]
  dst = edge_index[1]
  pad = E_PAD - E
  src1d = jnp.concatenate([src, jnp.zeros((pad,), src.dtype)]).astype(jnp.int32)
  dst1d = jnp.concatenate([dst, jnp.full((pad,), N, dst.dtype)]).astype(jnp.int32)
  dst2d = dst1d.reshape(E_PAD // EB, EB)
  ng = RPT // 8
  ix = jnp.concatenate([src1d.reshape(16, ng, 8, EB),
                        dst1d.reshape(16, ng, 8, EB)], axis=2).reshape(-1, EB)
  types2d = node_types.reshape(N, 1).astype(jnp.int32)

  x0, x1 = _embed(types2d, node_emb, type_emb)
  degf = _make_deg()(dst2d)
  dcol = degf[:N].reshape(N, 1)

  a0, a1 = _make_segsum(2)(ix, x0, x1)
  h = _sage0(a0, a1, x0, x1, dcol, W_l0, W_r0, b_l0.reshape(1, -1),
             bn_g0.reshape(1, -1), bn_b0.reshape(1, -1))
  b4 = _make_segsum(4)(ix, *h)
  h2 = _sage1(*b4, *h, dcol, W_l1, W_r1, b_l1.reshape(1, -1),
              bn_g1.reshape(1, -1), bn_b1.reshape(1, -1))
  u0, u1, v = _l2(*h2, W_l2, W_r2, b_l2.reshape(1, -1))
  c0, c1 = _make_segsum(2)(ix, u0, u1)
  return _fin(c0, c1, dcol, v)


# speedup vs baseline: 3.7875x; 1.0696x over previous
import functools
import numpy as np
import jax
import jax.numpy as jnp
from jax import lax
from jax.experimental import pallas as pl
from jax.experimental.pallas import tpu as pltpu
from jax.experimental.pallas import tpu_sc as plsc

N = 10000
E = 160000
NP = 10240
EB = 128
RPT = 80
EPT = RPT * EB
E_PAD = 16 * EPT
BM = 2000
NCORES = 2
NSUB = 16

_mesh = lambda: plsc.VectorSubcoreMesh(
    core_axis_name="c", subcore_axis_name="s", num_cores=NCORES, num_subcores=NSUB)


@functools.cache
def _make_segsum(nchunks):
  NG = RPT // 8

  def body(ix_hbm, *rest):
    feats = rest[:nchunks]
    outs = rest[nchunks:2 * nchunks]
    (ibuf, gb0, gb1, accum,
     ssem0, ssem1, gsem0, gsem1, isem) = rest[2 * nchunks:]
    gbufs = (gb0, gb1)
    ssems = (ssem0, ssem1)
    gsems = (gsem0, gsem1)
    core = lax.axis_index("c")
    t = lax.axis_index("s")

    for step in range(nchunks // 2):
      def zrow(i, _):
        for gq in range(8):
          gb0[i, pl.ds(gq * 16, 16)] = jnp.zeros((16,), jnp.float32)
        return 0
      lax.fori_loop(0, EB, zrow, 0)
      for k5 in range(5):
        pltpu.sync_copy(gb0, accum.at[pl.ds(t * 640 + k5 * EB, EB)])
      plsc.subcore_barrier()

      for k in range(2):
        @pl.when(core == k)
        def _(step=step, k=k):
          feat = feats[2 * step + k]
          pltpu.sync_copy(ix_hbm.at[pl.ds(t * NG * 16, 16)], ibuf.at[0])

          def gloop(g, _):
            @pl.when(g > 0)
            def _wi():
              pltpu.make_async_copy(
                  ix_hbm.at[pl.ds(0, 16)], ibuf.at[0], isem).wait()
            p = lax.rem(g, 2)
            for jj in range(4):
              for kk in range(2):
                b = g * 8 + jj * 2 + kk
                r = jj * 2 + kk
                @pl.when(b >= 2)
                def _ws(kk=kk):
                  pltpu.make_async_copy(
                      gbufs[kk], accum.at[ibuf.at[0, 8]], ssems[kk]).wait()
                pltpu.async_copy(feat.at[ibuf.at[p, r]], gbufs[kk],
                                 gsems[kk])
                prev = (ibuf.at[p, 8 + r - 1] if r > 0
                        else ibuf.at[1 - p, 15])
                @pl.when(b >= 1)
                def _sp(kk=kk, prev=prev):
                  pltpu.make_async_copy(
                      feat.at[ibuf.at[0, 0]], gbufs[1 - kk],
                      gsems[1 - kk]).wait()
                  pltpu.async_copy(gbufs[1 - kk], accum.at[prev],
                                   ssems[1 - kk], add=True)
              if jj == 0:
                @pl.when(g + 1 < NG)
                def _pf(g=g):
                  p2 = lax.rem(g + 1, 2)
                  pltpu.async_copy(
                      ix_hbm.at[pl.ds((t * NG + g + 1) * 16, 16)],
                      ibuf.at[p2], isem)
            return 0
          lax.fori_loop(0, NG, gloop, 0)
          pltpu.make_async_copy(
              feat.at[ibuf.at[0, 0]], gbufs[1], gsems[1]).wait()
          pltpu.async_copy(gbufs[1], accum.at[ibuf.at[1, 15]],
                           ssems[1], add=True)
          pltpu.make_async_copy(
              gbufs[0], accum.at[ibuf.at[0, 8]], ssems[0]).wait()
          pltpu.make_async_copy(
              gbufs[1], accum.at[ibuf.at[0, 8]], ssems[1]).wait()
      plsc.subcore_barrier()

      for k in range(2):
        @pl.when(core == k)
        def _(step=step, k=k):
          pltpu.sync_copy(accum.at[pl.ds(t * 640, 640)],
                          outs[2 * step + k].at[pl.ds(t * 640, 640)])
      if step + 1 < nchunks // 2:
        plsc.subcore_barrier()

  return pl.kernel(
      body,
      out_type=[jax.ShapeDtypeStruct((NP, 128), jnp.float32)] * nchunks,
      mesh=_mesh(),
      scratch_types=[
          pltpu.VMEM((2, 16, EB), jnp.int32),
          pltpu.VMEM((EB, 128), jnp.float32),
          pltpu.VMEM((EB, 128), jnp.float32),
          pltpu.VMEM_SHARED((NP, 128), jnp.float32),
          pltpu.SemaphoreType.DMA,
          pltpu.SemaphoreType.DMA,
          pltpu.SemaphoreType.DMA,
          pltpu.SemaphoreType.DMA,
          pltpu.SemaphoreType.DMA,
      ])


@functools.cache
def _make_deg():
  def body(dst_hbm, deg_hbm, dstv, degv, degout, vbuf, sp):
    core = lax.axis_index("c")
    t = lax.axis_index("s")
    pltpu.sync_copy(dst_hbm.at[pl.ds(t * RPT, RPT)], dstv)

    def z(i, _):
      degv[pl.ds(i * 16, 16)] = jnp.zeros((16,), jnp.float32)
      return 0
    lax.fori_loop(0, NP // 16, z, 0)

    ones = jnp.ones((16,), jnp.float32)
    def acc(r, _):
      for g in range(EB // 16):
        idx = dstv[r, pl.ds(g * 16, 16)]
        plsc.addupdate_scatter(degv, [idx], ones)
      return 0
    lax.fori_loop(0, RPT, acc, 0)

    pltpu.sync_copy(degv, sp.at[t])
    plsc.subcore_barrier()
    for r in range(16):
      pltpu.sync_copy(sp.at[r, pl.ds(t * 640, 640)], vbuf.at[r])

    def red(g, _):
      s = jnp.zeros((16,), jnp.float32)
      for r in range(16):
        s = s + vbuf[r, pl.ds(g * 16, 16)]
      degout[pl.ds(g * 16, 16)] = s
      return 0
    lax.fori_loop(0, 40, red, 0)

    @pl.when(core == 0)
    def _():
      pltpu.sync_copy(degout, deg_hbm.at[pl.ds(t * 640, 640)])

  return pl.kernel(
      body,
      out_type=jax.ShapeDtypeStruct((NP,), jnp.float32),
      mesh=_mesh(),
      compiler_params=pltpu.CompilerParams(needs_layout_passes=False),
      scratch_types=[
          pltpu.VMEM((RPT, EB), jnp.int32),
          pltpu.VMEM((NP,), jnp.float32),
          pltpu.VMEM((640,), jnp.float32),
          pltpu.VMEM((16, 640), jnp.float32),
          pltpu.VMEM_SHARED((16, NP), jnp.float32),
      ])


def _embed_body(types_ref, emb_ref, temb_ref, o0, o1):
  t = types_ref[...]
  oh = (t == lax.broadcasted_iota(jnp.int32, (1, 8), 1)).astype(jnp.float32)
  x = emb_ref[...] + jnp.dot(oh, temb_ref[...], preferred_element_type=jnp.float32)
  o0[...] = x[:, :128]
  o1[...] = x[:, 128:]


_embed = pl.pallas_call(
    _embed_body,
    grid=(N // BM,),
    in_specs=[
        pl.BlockSpec((BM, 1), lambda i: (i, 0)),
        pl.BlockSpec((BM, 256), lambda i: (i, 0)),
        pl.BlockSpec((8, 256), lambda i: (0, 0)),
    ],
    out_specs=[pl.BlockSpec((BM, 128), lambda i: (i, 0))] * 2,
    out_shape=[jax.ShapeDtypeStruct((N, 128), jnp.float32)] * 2)


def _make_sage(cin, cout):
  dout = cout * 128
  def body(*refs):
    aggs = refs[0:cin]
    xs = refs[cin:2 * cin]
    dref, Wl, Wr, bl, g, bb = refs[2 * cin:2 * cin + 6]
    outs = refs[2 * cin + 6:]
    inv = 1.0 / jnp.maximum(dref[...], 1.0)
    acc = jnp.zeros((BM, dout), jnp.float32)
    for c in range(cin):
      acc = acc + jnp.dot(aggs[c][...] * inv, Wl[c * 128:(c + 1) * 128, :],
                          preferred_element_type=jnp.float32)
      acc = acc + jnp.dot(xs[c][...], Wr[c * 128:(c + 1) * 128, :],
                          preferred_element_type=jnp.float32)
    s = g[...] * np.float32(1.0 / np.sqrt(1.0 + 1e-5))
    h = jnp.maximum(acc * s + (bl[...] * s + bb[...]), 0.0)
    for c in range(cout):
      outs[c][...] = h[:, c * 128:(c + 1) * 128]

  din = cin * 128
  return pl.pallas_call(
      body,
      grid=(N // BM,),
      in_specs=(
          [pl.BlockSpec((BM, 128), lambda i: (i, 0))] * (2 * cin) +
          [pl.BlockSpec((BM, 1), lambda i: (i, 0)),
           pl.BlockSpec((din, dout), lambda i: (0, 0)),
           pl.BlockSpec((din, dout), lambda i: (0, 0)),
           pl.BlockSpec((1, dout), lambda i: (0, 0)),
           pl.BlockSpec((1, dout), lambda i: (0, 0)),
           pl.BlockSpec((1, dout), lambda i: (0, 0))]),
      out_specs=[pl.BlockSpec((BM, 128), lambda i: (i, 0))] * cout,
      out_shape=[jax.ShapeDtypeStruct((N, 128), jnp.float32)] * cout)


def _l2_body(x0, x1, x2, x3, Wl, Wr, bl, u0, u1, v):
  xs = (x0, x1, x2, x3)
  u = jnp.zeros((BM, 256), jnp.float32)
  vv = jnp.zeros((BM, 256), jnp.float32)
  for c in range(4):
    u = u + jnp.dot(xs[c][...], Wl[c * 128:(c + 1) * 128, :],
                    preferred_element_type=jnp.float32)
    vv = vv + jnp.dot(xs[c][...], Wr[c * 128:(c + 1) * 128, :],
                      preferred_element_type=jnp.float32)
  vv = vv + bl[...]
  u0[...] = u[:, :128]
  u1[...] = u[:, 128:]
  v[...] = vv


_l2 = pl.pallas_call(
    _l2_body,
    grid=(N // BM,),
    in_specs=(
        [pl.BlockSpec((BM, 128), lambda i: (i, 0))] * 4 +
        [pl.BlockSpec((512, 256), lambda i: (0, 0))] * 2 +
        [pl.BlockSpec((1, 256), lambda i: (0, 0))]),
    out_specs=[pl.BlockSpec((BM, 128), lambda i: (i, 0))] * 2 +
              [pl.BlockSpec((BM, 256), lambda i: (i, 0))],
    out_shape=[jax.ShapeDtypeStruct((N, 128), jnp.float32)] * 2 +
              [jax.ShapeDtypeStruct((N, 256), jnp.float32)])


def _fin_body(a0, a1, dref, vref, o):
  inv = 1.0 / jnp.maximum(dref[...], 1.0)
  o[...] = jnp.concatenate([a0[...] * inv, a1[...] * inv], axis=1) + vref[...]


_fin = pl.pallas_call(
    _fin_body,
    grid=(N // BM,),
    in_specs=[
        pl.BlockSpec((BM, 128), lambda i: (i, 0)),
        pl.BlockSpec((BM, 128), lambda i: (i, 0)),
        pl.BlockSpec((BM, 1), lambda i: (i, 0)),
        pl.BlockSpec((BM, 256), lambda i: (i, 0)),
    ],
    out_specs=pl.BlockSpec((BM, 256), lambda i: (i, 0)),
    out_shape=jax.ShapeDtypeStruct((N, 256), jnp.float32))


_sage0 = _make_sage(2, 4)
_sage1 = _make_sage(4, 4)


def kernel(edge_index, node_types, node_emb, type_emb, W_l0, b_l0, W_r0,
           bn_g0, bn_b0, W_l1, b_l1, W_r1, bn_g1, bn_b1, W_l2, b_l2, W_r2):
  src = edge_index[0]
  dst = edge_index[1]
  pad = E_PAD - E
  src1d = jnp.concatenate([src, jnp.zeros((pad,), src.dtype)]).astype(jnp.int32)
  dst1d = jnp.concatenate([dst, jnp.full((pad,), N, dst.dtype)]).astype(jnp.int32)
  dst2d = dst1d.reshape(E_PAD // EB, EB)
  ng = RPT // 8
  ix = jnp.concatenate([src1d.reshape(16, ng, 8, EB),
                        dst1d.reshape(16, ng, 8, EB)], axis=2).reshape(-1, EB)
  types2d = node_types.reshape(N, 1).astype(jnp.int32)

  x0, x1 = _embed(types2d, node_emb, type_emb)
  degf = _make_deg()(dst2d)
  dcol = degf[:N].reshape(N, 1)

  a0, a1 = _make_segsum(2)(ix, x0, x1)
  h = _sage0(a0, a1, x0, x1, dcol, W_l0, W_r0, b_l0.reshape(1, -1),
             bn_g0.reshape(1, -1), bn_b0.reshape(1, -1))
  b4 = _make_segsum(4)(ix, *h)
  h2 = _sage1(*b4, *h, dcol, W_l1, W_r1, b_l1.reshape(1, -1),
              bn_g1.reshape(1, -1), bn_b1.reshape(1, -1))
  u0, u1, v = _l2(*h2, W_l2, W_r2, b_l2.reshape(1, -1))
  c0, c1 = _make_segsum(2)(ix, u0, u1)
  return _fin(c0, c1, dcol, v)
